# baseline (speedup 1.0000x reference)
import jax
import jax.numpy as jnp
from jax.experimental import pallas as pl
from jax.experimental.pallas import tpu as pltpu

_F32 = jnp.float32
_BF16 = jnp.bfloat16


def _pick_block(n, prefs):
    for r in prefs:
        if n % r == 0 and r <= n:
            return r
    return n


def _bdot(a, b):
    return jnp.dot(a.astype(_BF16), b.astype(_BF16), preferred_element_type=_F32)


def _prep_body(o1_ref, o2_ref, w11_ref, b11_ref, w12_ref, b12_ref,
               w21_ref, b21_ref, w22_ref, b22_ref, e1_ref, e2_ref,
               z1_ref, z2_ref):
    f1 = _bdot(o1_ref[...], w11_ref[...]) + b11_ref[...]
    f1 = _bdot(f1, w12_ref[...]) + b12_ref[...]
    z1_ref[...] = _bdot(f1, e1_ref[...])
    f2 = _bdot(o2_ref[...], w21_ref[...]) + b21_ref[...]
    f2 = _bdot(f2, w22_ref[...]) + b22_ref[...]
    z2_ref[...] = _bdot(f2, e2_ref[...])


def _enc_body(adj1_ref, adj2_ref, z1_ref, z2_ref, wom_ref, u_ref,
              d1w_ref, d2w_ref,
              emb1_ref, emb2_ref, embc_ref, alpha_ref, y1_ref, y2_ref):
    e1 = _bdot(adj1_ref[...], z1_ref[...])
    e2 = _bdot(adj2_ref[...], z2_ref[...])
    emb1_ref[...] = e1
    emb2_ref[...] = e2
    wom = wom_ref[...]
    v1 = jnp.tanh(_bdot(e1, wom))
    v2 = jnp.tanh(_bdot(e2, wom))
    u = u_ref[...].astype(_BF16).astype(_F32)
    vu1 = jnp.sum(v1.astype(_BF16).astype(_F32) * u, axis=1, keepdims=True)
    vu2 = jnp.sum(v2.astype(_BF16).astype(_F32) * u, axis=1, keepdims=True)
    vu1 = vu1 + 1e-6
    vu2 = vu2 + 1e-6
    m = jnp.maximum(vu1, vu2)
    x1 = jnp.exp(vu1 - m)
    x2 = jnp.exp(vu2 - m)
    s = x1 + x2
    a1 = x1 / s
    a2 = x2 / s
    ec = a1 * e1 + a2 * e2
    embc_ref[...] = ec
    alpha_ref[...] = jnp.concatenate([a1, a2], axis=1)
    ecb = ec
    y1_ref[...] = _bdot(ecb, d1w_ref[...])
    y2_ref[...] = _bdot(ecb, d2w_ref[...])


def _dec_body(adj1_ref, adj2_ref, y1_ref, y2_ref,
              w31_ref, b31_ref, w32_ref, b32_ref,
              w41_ref, b41_ref, w42_ref, b42_ref,
              f1_ref, f2_ref):
    d1 = _bdot(adj1_ref[...], y1_ref[...])
    h1 = _bdot(d1, w31_ref[...]) + b31_ref[...]
    f1_ref[...] = _bdot(h1, w32_ref[...]) + b32_ref[...]
    d2 = _bdot(adj2_ref[...], y2_ref[...])
    h2 = _bdot(d2, w41_ref[...]) + b41_ref[...]
    f2_ref[...] = _bdot(h2, w42_ref[...]) + b42_ref[...]


def _full(shape):
    return pl.BlockSpec(shape, lambda i: (0,) * len(shape))


def kernel(omic1, omic2, adj_1, adj_2, mlp1_w1, mlp1_b1, mlp1_w2, mlp1_b2,
           mlp2_w1, mlp2_b1, mlp2_w2, mlp2_b2, enc1_w, enc2_w, w_omega,
           u_omega, dec1_w, dec2_w, mlp3_w1, mlp3_b1, mlp3_w2, mlp3_b2,
           mlp4_w1, mlp4_b1, mlp4_w2, mlp4_b2):
    n, ni1 = omic1.shape
    ni2 = omic2.shape[1]
    nh1, no1 = mlp1_w2.shape[0], mlp1_w2.shape[1]
    nh2, no2 = mlp2_w2.shape[0], mlp2_w2.shape[1]
    d = enc1_w.shape[1]

    params = pltpu.CompilerParams(dimension_semantics=("parallel",))

    rp = _pick_block(n, (1000, 500, 200, 100, 8))
    np_ = n // rp
    z1, z2 = pl.pallas_call(
        _prep_body,
        grid=(np_,),
        in_specs=[
            pl.BlockSpec((rp, ni1), lambda i: (i, 0)),
            pl.BlockSpec((rp, ni2), lambda i: (i, 0)),
            _full((ni1, nh1)), _full((1, nh1)),
            _full((nh1, no1)), _full((1, no1)),
            _full((ni2, nh2)), _full((1, nh2)),
            _full((nh2, no2)), _full((1, no2)),
            _full((no1, d)), _full((no2, d)),
        ],
        out_specs=[
            pl.BlockSpec((rp, d), lambda i: (i, 0)),
            pl.BlockSpec((rp, d), lambda i: (i, 0)),
        ],
        out_shape=[
            jax.ShapeDtypeStruct((n, d), _F32),
            jax.ShapeDtypeStruct((n, d), _F32),
        ],
        compiler_params=params,
    )(omic1, omic2,
      mlp1_w1, mlp1_b1.reshape(1, -1), mlp1_w2, mlp1_b2.reshape(1, -1),
      mlp2_w1, mlp2_b1.reshape(1, -1), mlp2_w2, mlp2_b2.reshape(1, -1),
      enc1_w, enc2_w)

    re = _pick_block(n, (80, 40, 8))
    ne = n // re
    emb1, emb2, embc, alpha, y1, y2 = pl.pallas_call(
        _enc_body,
        grid=(ne,),
        in_specs=[
            pl.BlockSpec((re, n), lambda i: (i, 0)),
            pl.BlockSpec((re, n), lambda i: (i, 0)),
            _full((n, d)), _full((n, d)),
            _full((d, d)), _full((1, d)),
            _full((d, no1)), _full((d, no2)),
        ],
        out_specs=[
            pl.BlockSpec((re, d), lambda i: (i, 0)),
            pl.BlockSpec((re, d), lambda i: (i, 0)),
            pl.BlockSpec((re, d), lambda i: (i, 0)),
            pl.BlockSpec((re, 2), lambda i: (i, 0)),
            pl.BlockSpec((re, no1), lambda i: (i, 0)),
            pl.BlockSpec((re, no2), lambda i: (i, 0)),
        ],
        out_shape=[
            jax.ShapeDtypeStruct((n, d), _F32),
            jax.ShapeDtypeStruct((n, d), _F32),
            jax.ShapeDtypeStruct((n, d), _F32),
            jax.ShapeDtypeStruct((n, 2), _F32),
            jax.ShapeDtypeStruct((n, no1), _F32),
            jax.ShapeDtypeStruct((n, no2), _F32),
        ],
        compiler_params=params,
    )(adj_1, adj_2, z1, z2, w_omega, u_omega.reshape(1, -1), dec1_w, dec2_w)

    f1a, f2a = pl.pallas_call(
        _dec_body,
        grid=(ne,),
        in_specs=[
            pl.BlockSpec((re, n), lambda i: (i, 0)),
            pl.BlockSpec((re, n), lambda i: (i, 0)),
            _full((n, no1)), _full((n, no2)),
            _full((no1, nh1)), _full((1, nh1)),
            _full((nh1, ni1)), _full((1, ni1)),
            _full((no2, nh2)), _full((1, nh2)),
            _full((nh2, ni2)), _full((1, ni2)),
        ],
        out_specs=[
            pl.BlockSpec((re, ni1), lambda i: (i, 0)),
            pl.BlockSpec((re, ni2), lambda i: (i, 0)),
        ],
        out_shape=[
            jax.ShapeDtypeStruct((n, ni1), _F32),
            jax.ShapeDtypeStruct((n, ni2), _F32),
        ],
        compiler_params=params,
    )(adj_1, adj_2, y1, y2,
      mlp3_w1, mlp3_b1.reshape(1, -1), mlp3_w2, mlp3_b2.reshape(1, -1),
      mlp4_w1, mlp4_b1.reshape(1, -1), mlp4_w2, mlp4_b2.reshape(1, -1))

    return (emb1, emb2, embc, alpha, f1a, f2a, emb1, emb2)


# speedup vs baseline: 1.1570x; 1.1570x over previous
import jax
import jax.numpy as jnp
from jax.experimental import pallas as pl
from jax.experimental.pallas import tpu as pltpu

_F32 = jnp.float32
_BF16 = jnp.bfloat16


def _pick_block(n, prefs):
    for r in prefs:
        if n % r == 0 and r <= n:
            return r
    return n


def _bdot(a, b):
    return jnp.dot(a.astype(_BF16), b.astype(_BF16), preferred_element_type=_F32)


def _prep_body(o1_ref, o2_ref, w11_ref, b11_ref, w12_ref, b12_ref,
               w21_ref, b21_ref, w22_ref, b22_ref, e1_ref, e2_ref,
               z1_ref, z2_ref):
    f1 = _bdot(o1_ref[...], w11_ref[...]) + b11_ref[...]
    f1 = _bdot(f1, w12_ref[...]) + b12_ref[...]
    z1_ref[...] = _bdot(f1, e1_ref[...])
    f2 = _bdot(o2_ref[...], w21_ref[...]) + b21_ref[...]
    f2 = _bdot(f2, w22_ref[...]) + b22_ref[...]
    z2_ref[...] = _bdot(f2, e2_ref[...])


def _enc_body(adj1_ref, adj2_ref, z1_ref, z2_ref, wom_ref, u_ref,
              d1w_ref, d2w_ref,
              emb1_ref, emb2_ref, embc_ref, alpha_ref, y1_ref, y2_ref):
    e1 = _bdot(adj1_ref[...], z1_ref[...])
    e2 = _bdot(adj2_ref[...], z2_ref[...])
    emb1_ref[...] = e1
    emb2_ref[...] = e2
    wom = wom_ref[...]
    v1 = jnp.tanh(_bdot(e1, wom))
    v2 = jnp.tanh(_bdot(e2, wom))
    u = u_ref[...].astype(_BF16).astype(_F32)
    vu1 = jnp.sum(v1.astype(_BF16).astype(_F32) * u, axis=1, keepdims=True)
    vu2 = jnp.sum(v2.astype(_BF16).astype(_F32) * u, axis=1, keepdims=True)
    vu1 = vu1 + 1e-6
    vu2 = vu2 + 1e-6
    m = jnp.maximum(vu1, vu2)
    x1 = jnp.exp(vu1 - m)
    x2 = jnp.exp(vu2 - m)
    s = x1 + x2
    a1 = x1 / s
    a2 = x2 / s
    ec = a1 * e1 + a2 * e2
    embc_ref[...] = ec
    alpha_ref[...] = jnp.concatenate([a1, a2], axis=1)
    ecb = ec
    y1_ref[...] = _bdot(ecb, d1w_ref[...])
    y2_ref[...] = _bdot(ecb, d2w_ref[...])


def _dec_body(adj1_ref, adj2_ref, y1_ref, y2_ref,
              w31_ref, b31_ref, w32_ref, b32_ref,
              w41_ref, b41_ref, w42_ref, b42_ref,
              f1_ref, f2_ref):
    d1 = _bdot(adj1_ref[...], y1_ref[...])
    h1 = _bdot(d1, w31_ref[...]) + b31_ref[...]
    f1_ref[...] = _bdot(h1, w32_ref[...]) + b32_ref[...]
    d2 = _bdot(adj2_ref[...], y2_ref[...])
    h2 = _bdot(d2, w41_ref[...]) + b41_ref[...]
    f2_ref[...] = _bdot(h2, w42_ref[...]) + b42_ref[...]


def _full(shape):
    return pl.BlockSpec(shape, lambda i: (0,) * len(shape))


def kernel(omic1, omic2, adj_1, adj_2, mlp1_w1, mlp1_b1, mlp1_w2, mlp1_b2,
           mlp2_w1, mlp2_b1, mlp2_w2, mlp2_b2, enc1_w, enc2_w, w_omega,
           u_omega, dec1_w, dec2_w, mlp3_w1, mlp3_b1, mlp3_w2, mlp3_b2,
           mlp4_w1, mlp4_b1, mlp4_w2, mlp4_b2):
    n, ni1 = omic1.shape
    ni2 = omic2.shape[1]
    nh1, no1 = mlp1_w2.shape[0], mlp1_w2.shape[1]
    nh2, no2 = mlp2_w2.shape[0], mlp2_w2.shape[1]
    d = enc1_w.shape[1]

    params = pltpu.CompilerParams(dimension_semantics=("parallel",))

    rp = _pick_block(n, (1000, 500, 200, 100, 8))
    np_ = n // rp
    z1, z2 = pl.pallas_call(
        _prep_body,
        grid=(np_,),
        in_specs=[
            pl.BlockSpec((rp, ni1), lambda i: (i, 0)),
            pl.BlockSpec((rp, ni2), lambda i: (i, 0)),
            _full((ni1, nh1)), _full((1, nh1)),
            _full((nh1, no1)), _full((1, no1)),
            _full((ni2, nh2)), _full((1, nh2)),
            _full((nh2, no2)), _full((1, no2)),
            _full((no1, d)), _full((no2, d)),
        ],
        out_specs=[
            pl.BlockSpec((rp, d), lambda i: (i, 0)),
            pl.BlockSpec((rp, d), lambda i: (i, 0)),
        ],
        out_shape=[
            jax.ShapeDtypeStruct((n, d), _F32),
            jax.ShapeDtypeStruct((n, d), _F32),
        ],
        compiler_params=params,
    )(omic1, omic2,
      mlp1_w1, mlp1_b1.reshape(1, -1), mlp1_w2, mlp1_b2.reshape(1, -1),
      mlp2_w1, mlp2_b1.reshape(1, -1), mlp2_w2, mlp2_b2.reshape(1, -1),
      enc1_w, enc2_w)

    re = _pick_block(n, (200, 100, 40, 8))
    ne = n // re
    emb1, emb2, embc, alpha, y1, y2 = pl.pallas_call(
        _enc_body,
        grid=(ne,),
        in_specs=[
            pl.BlockSpec((re, n), lambda i: (i, 0)),
            pl.BlockSpec((re, n), lambda i: (i, 0)),
            _full((n, d)), _full((n, d)),
            _full((d, d)), _full((1, d)),
            _full((d, no1)), _full((d, no2)),
        ],
        out_specs=[
            pl.BlockSpec((re, d), lambda i: (i, 0)),
            pl.BlockSpec((re, d), lambda i: (i, 0)),
            pl.BlockSpec((re, d), lambda i: (i, 0)),
            pl.BlockSpec((re, 2), lambda i: (i, 0)),
            pl.BlockSpec((re, no1), lambda i: (i, 0)),
            pl.BlockSpec((re, no2), lambda i: (i, 0)),
        ],
        out_shape=[
            jax.ShapeDtypeStruct((n, d), _F32),
            jax.ShapeDtypeStruct((n, d), _F32),
            jax.ShapeDtypeStruct((n, d), _F32),
            jax.ShapeDtypeStruct((n, 2), _F32),
            jax.ShapeDtypeStruct((n, no1), _F32),
            jax.ShapeDtypeStruct((n, no2), _F32),
        ],
        compiler_params=params,
    )(adj_1, adj_2, z1, z2, w_omega, u_omega.reshape(1, -1), dec1_w, dec2_w)

    f1a, f2a = pl.pallas_call(
        _dec_body,
        grid=(ne,),
        in_specs=[
            pl.BlockSpec((re, n), lambda i: (i, 0)),
            pl.BlockSpec((re, n), lambda i: (i, 0)),
            _full((n, no1)), _full((n, no2)),
            _full((no1, nh1)), _full((1, nh1)),
            _full((nh1, ni1)), _full((1, ni1)),
            _full((no2, nh2)), _full((1, nh2)),
            _full((nh2, ni2)), _full((1, ni2)),
        ],
        out_specs=[
            pl.BlockSpec((re, ni1), lambda i: (i, 0)),
            pl.BlockSpec((re, ni2), lambda i: (i, 0)),
        ],
        out_shape=[
            jax.ShapeDtypeStruct((n, ni1), _F32),
            jax.ShapeDtypeStruct((n, ni2), _F32),
        ],
        compiler_params=params,
    )(adj_1, adj_2, y1, y2,
      mlp3_w1, mlp3_b1.reshape(1, -1), mlp3_w2, mlp3_b2.reshape(1, -1),
      mlp4_w1, mlp4_b1.reshape(1, -1), mlp4_w2, mlp4_b2.reshape(1, -1))

    return (emb1, emb2, embc, alpha, f1a, f2a, emb1, emb2)
